# baseline (speedup 1.0000x reference)
import functools
import math

import jax
import jax.numpy as jnp
from jax import lax
from jax.experimental import pallas as pl
from jax.experimental.pallas import tpu as pltpu
from jax.experimental.pallas import tpu_sc as plsc

SEQ_LEN = 8192
DIM = 64
VOCAB = 1000000
_GRAN = 16
_GPD = VOCAB // _GRAN
_NC, _NS = 2, 16
_NW = _NC * _NS
_D_PER_W = DIM // _NW
_CHUNK = 2048
_N_CHUNKS = SEQ_LEN // _CHUNK


def _positional_encoding_t():
    position = jnp.arange(SEQ_LEN, dtype=jnp.float32)[None, :]
    div_term = jnp.exp(
        jnp.arange(0, DIM, 2, dtype=jnp.float32) * (-math.log(10000.0) / DIM)
    )
    pe = jnp.zeros((DIM, SEQ_LEN), dtype=jnp.float32)
    pe = pe.at[0::2, :].set(jnp.sin(div_term[:, None] * position))
    pe = pe.at[1::2, :].set(jnp.cos(div_term[:, None] * position))
    return pe


_mesh = plsc.VectorSubcoreMesh(core_axis_name="c", subcore_axis_name="s")


@functools.partial(
    pl.kernel,
    mesh=_mesh,
    out_type=jax.ShapeDtypeStruct((DIM, SEQ_LEN), jnp.float32),
    scratch_types=[
        pltpu.VMEM((SEQ_LEN,), jnp.int32),
        pltpu.VMEM((SEQ_LEN,), jnp.int32),
        pltpu.VMEM((_CHUNK,), jnp.int32),
        pltpu.VMEM((_CHUNK, _GRAN), jnp.float32),
        pltpu.VMEM((_CHUNK,), jnp.float32),
        pltpu.SemaphoreType.DMA,
    ],
    compiler_params=pltpu.CompilerParams(
        use_tc_tiling_on_sc=False, needs_layout_passes=False
    ),
)
def _sc_gather_t(
    tab16_hbm, gidx_hbm, lid_hbm, out_hbm,
    gidx_v, lid_v, bidx_v, buf_v, cbuf_v, sem,
):
    wid = lax.axis_index("s") * _NC + lax.axis_index("c")
    d0 = wid * _D_PER_W
    pltpu.sync_copy(gidx_hbm, gidx_v)
    pltpu.sync_copy(lid_hbm, lid_v)

    @pl.loop(0, _D_PER_W)
    def _(r):
        d = d0 + r

        @pl.loop(0, _N_CHUNKS)
        def _(c):
            base = c * _CHUNK

            @pl.loop(0, _CHUNK, step=_GRAN)
            def _(j):
                bidx_v[pl.ds(j, _GRAN)] = gidx_v[pl.ds(base + j, _GRAN)] + d * _GPD

            pltpu.async_copy(tab16_hbm.at[bidx_v], buf_v, sem).wait()

            @pl.loop(0, _CHUNK, step=_GRAN)
            def _(j):
                rid = lax.broadcasted_iota(jnp.int32, (_GRAN,), 0) + j
                lid = lid_v[pl.ds(base + j, _GRAN)]
                cbuf_v[pl.ds(j, _GRAN)] = plsc.load_gather(buf_v, [rid, lid])

            pltpu.sync_copy(cbuf_v, out_hbm.at[d, pl.ds(base, _CHUNK)])


def _tc_add(x_ref, pe_ref, o_ref):
    o_ref[...] = x_ref[...] + pe_ref[...]


def kernel(indices, table):
    idx = indices.astype(jnp.int32)
    tab16 = table.T.reshape(DIM * _GPD, _GRAN)
    gathered_t = _sc_gather_t(tab16, idx >> 4, idx & (_GRAN - 1))
    pe_t = _positional_encoding_t()
    out_t = pl.pallas_call(
        _tc_add,
        out_shape=jax.ShapeDtypeStruct((DIM, SEQ_LEN), jnp.float32),
    )(gathered_t, pe_t)
    return out_t.T[None, :, :]
